# baseline (speedup 1.0000x reference)
import functools

import jax
import jax.numpy as jnp
from jax import lax
from jax.experimental import pallas as pl
from jax.experimental.pallas import tpu as pltpu
from jax.experimental.pallas import tpu_sc as plsc

_N_EDGES = 320000
_D = 128
_N_SEG = 10000
_NC = 2
_NS = 16
_NW = _NC * _NS
_EDGES_PER_TILE = _N_EDGES // _NW
_CHUNK = 128
_NFULL = _EDGES_PER_TILE // _CHUNK
_TAIL = _EDGES_PER_TILE - _NFULL * _CHUNK
_NBUF = 3
_SEG_PER_TILE = _N_SEG // _NS
_WB_CHUNK = 16
_N_WB_CHUNKS = _N_SEG // _WB_CHUNK


def _sc_body(x_hbm, idx_hbm, out_hbm, acc_sh, idx_v, idx_t, rows_v, *sems):
    row_sems = sems[:_NBUF]
    idx_sems = sems[_NBUF:2 * _NBUF]
    scat_sems = sems[2 * _NBUF:3 * _NBUF]
    tail_sem = sems[3 * _NBUF]
    c = lax.axis_index("c")
    s = lax.axis_index("s")
    tid = c * _NS + s
    base = tid * _EDGES_PER_TILE

    def _row_src(ci):
        off = pl.multiple_of(base + ci * _CHUNK, 16)
        return x_hbm.at[pl.ds(off, _CHUNK)]

    def _idx_src(ci):
        off = pl.multiple_of(base + ci * _CHUNK, 16)
        return idx_hbm.at[pl.ds(off, _CHUNK)]

    def _start_loads(ci, b):
        pltpu.async_copy(_row_src(ci), rows_v.at[b], row_sems[b])
        pltpu.async_copy(_idx_src(ci), idx_v.at[b], idx_sems[b])

    def _wait_loads(ci, b):
        pltpu.make_async_copy(_row_src(ci), rows_v.at[b], row_sems[b]).wait()
        pltpu.make_async_copy(_idx_src(ci), idx_v.at[b],
                              idx_sems[b]).wait()

    for b in range(_NBUF - 1):
        _start_loads(b, b)

    zb = _NBUF - 1
    zvec = jnp.zeros((16,), jnp.float32)
    def _zero_row(i, _):
        def _zero_lane(k, _):
            rows_v[zb, i, pl.ds(k * 16, 16)] = zvec
            return ()
        lax.fori_loop(0, _D // 16, _zero_lane, (), unroll=True)
        return ()
    lax.fori_loop(0, _CHUNK, _zero_row, ())
    seg0 = s * _SEG_PER_TILE
    for j in range(_SEG_PER_TILE // _CHUNK):
        pltpu.sync_copy(rows_v.at[zb],
                        acc_sh.at[pl.ds(seg0 + j * _CHUNK, _CHUNK)])
    rem = _SEG_PER_TILE % _CHUNK
    pltpu.sync_copy(rows_v.at[zb, pl.ds(0, rem)],
                    acc_sh.at[pl.ds(seg0 + _SEG_PER_TILE - rem, rem)])
    _start_loads(zb, zb)
    plsc.subcore_barrier()

    def _scatter(b):
        pltpu.async_copy(rows_v.at[b], acc_sh.at[idx_v.at[b]], scat_sems[b],
                         add=True)

    def _wait_scatter(b):
        pltpu.make_async_copy(rows_v.at[b], acc_sh.at[idx_v.at[b]],
                              scat_sems[b]).wait()

    _wait_loads(0, 0)
    _scatter(0)
    def _outer(jo, _):
        for b in range(_NBUF):
            k = jo * _NBUF + b + 1
            bk = (b + 1) % _NBUF
            _wait_loads(k, bk)
            _scatter(bk)
            bp = b % _NBUF
            _wait_scatter(bp)
            _start_loads(k + 2, bp)
        return ()
    lax.fori_loop(0, (_NFULL - 3) // _NBUF, _outer, ())
    for k in (_NFULL - 2, _NFULL - 1):
        bk = k % _NBUF
        _wait_loads(k, bk)
        _scatter(bk)
    for b in range(_NBUF):
        _wait_scatter(b)

    tail_off = pl.multiple_of(base + _NFULL * _CHUNK, 16)
    pltpu.sync_copy(idx_hbm.at[pl.ds(tail_off, _TAIL)], idx_t)
    pltpu.async_copy(x_hbm.at[pl.ds(tail_off, _TAIL)],
                     rows_v.at[0, pl.ds(0, _TAIL)], tail_sem)
    pltpu.make_async_copy(x_hbm.at[pl.ds(tail_off, _TAIL)],
                          rows_v.at[0, pl.ds(0, _TAIL)], tail_sem).wait()
    pltpu.sync_copy(rows_v.at[0, pl.ds(0, _TAIL)], acc_sh.at[idx_t],
                    add=True)
    plsc.subcore_barrier()

    def _wb(j, _):
        cw = j * _NS + s
        r0 = pl.multiple_of(cw * _WB_CHUNK, 16)
        pltpu.sync_copy(acc_sh.at[pl.ds(r0, _WB_CHUNK)],
                        out_hbm.at[c, pl.ds(r0, _WB_CHUNK)])
        return ()
    lax.fori_loop(0, _N_WB_CHUNKS // _NS, _wb, ())
    @pl.when(s == 0)
    def _():
        r0 = (_N_WB_CHUNKS // _NS) * _NS * _WB_CHUNK
        pltpu.sync_copy(acc_sh.at[pl.ds(r0, _WB_CHUNK)],
                        out_hbm.at[c, pl.ds(r0, _WB_CHUNK)])


def _tc_add(a_ref, b_ref, o_ref):
    o_ref[...] = a_ref[0] + b_ref[0]


@jax.jit
def kernel(x, index):
    idx = index.astype(jnp.int32)
    mesh = plsc.VectorSubcoreMesh(core_axis_name="c", subcore_axis_name="s")
    partials = pl.kernel(
        _sc_body,
        out_type=jax.ShapeDtypeStruct((_NC, _N_SEG, _D), jnp.float32),
        mesh=mesh,
        scratch_types=[
            pltpu.VMEM_SHARED((_N_SEG, _D), jnp.float32),
            pltpu.VMEM((_NBUF, _CHUNK), jnp.int32),
            pltpu.VMEM((_TAIL,), jnp.int32),
            pltpu.VMEM((_NBUF, _CHUNK, _D), jnp.float32),
            *([pltpu.SemaphoreType.DMA] * (3 * _NBUF + 1)),
        ],
    )(x, idx)

    blk = 2000
    out = pl.pallas_call(
        _tc_add,
        grid=(_N_SEG // blk,),
        in_specs=[
            pl.BlockSpec((1, blk, _D), lambda i: (0, i, 0)),
            pl.BlockSpec((1, blk, _D), lambda i: (1, i, 0)),
        ],
        out_specs=pl.BlockSpec((blk, _D), lambda i: (i, 0)),
        out_shape=jax.ShapeDtypeStruct((_N_SEG, _D), jnp.float32),
    )(partials, partials)
    return out


# speedup vs baseline: 1.2382x; 1.2382x over previous
import functools

import jax
import jax.numpy as jnp
from jax import lax
from jax.experimental import pallas as pl
from jax.experimental.pallas import tpu as pltpu
from jax.experimental.pallas import tpu_sc as plsc

_N_EDGES = 320000
_D = 128
_N_SEG = 10000
_NC = 2
_NS = 16
_NW = _NC * _NS
_EDGES_PER_TILE = _N_EDGES // _NW
_CHUNK = 128
_NFULL = _EDGES_PER_TILE // _CHUNK
_TAIL = _EDGES_PER_TILE - _NFULL * _CHUNK
_NBUF = 3
_SEG_PER_TILE = _N_SEG // _NS
_WB_CHUNK = 64
_N_WB_FULL = _N_SEG // _WB_CHUNK


def _sc_body(x_hbm, idx_hbm, out_hbm, acc_sh, idx_v, idx_t, rows_v, *sems):
    row_sems = sems[:_NBUF]
    idx_sems = sems[_NBUF:2 * _NBUF]
    scat_sems = sems[2 * _NBUF:3 * _NBUF]
    tail_sem = sems[3 * _NBUF]
    c = lax.axis_index("c")
    s = lax.axis_index("s")
    tid = c * _NS + s
    base = tid * _EDGES_PER_TILE

    def _row_src(ci):
        off = pl.multiple_of(base + ci * _CHUNK, 16)
        return x_hbm.at[pl.ds(off, _CHUNK)]

    def _idx_src(ci):
        off = pl.multiple_of(base + ci * _CHUNK, 16)
        return idx_hbm.at[pl.ds(off, _CHUNK)]

    def _start_loads(ci, b):
        pltpu.async_copy(_row_src(ci), rows_v.at[b], row_sems[b])
        pltpu.async_copy(_idx_src(ci), idx_v.at[b], idx_sems[b])

    def _wait_loads(ci, b):
        pltpu.make_async_copy(_row_src(ci), rows_v.at[b], row_sems[b]).wait()
        pltpu.make_async_copy(_idx_src(ci), idx_v.at[b],
                              idx_sems[b]).wait()

    for b in range(_NBUF - 1):
        _start_loads(b, b)

    zb = _NBUF - 1
    zvec = jnp.zeros((16,), jnp.float32)
    def _zero_row(i, _):
        def _zero_lane(k, _):
            rows_v[zb, i, pl.ds(k * 16, 16)] = zvec
            return ()
        lax.fori_loop(0, _D // 16, _zero_lane, (), unroll=True)
        return ()
    lax.fori_loop(0, _CHUNK, _zero_row, ())
    seg0 = s * _SEG_PER_TILE
    for j in range(_SEG_PER_TILE // _CHUNK):
        pltpu.sync_copy(rows_v.at[zb],
                        acc_sh.at[pl.ds(seg0 + j * _CHUNK, _CHUNK)])
    rem = _SEG_PER_TILE % _CHUNK
    pltpu.sync_copy(rows_v.at[zb, pl.ds(0, rem)],
                    acc_sh.at[pl.ds(seg0 + _SEG_PER_TILE - rem, rem)])
    _start_loads(zb, zb)
    plsc.subcore_barrier()

    def _scatter(b):
        pltpu.async_copy(rows_v.at[b], acc_sh.at[idx_v.at[b]], scat_sems[b],
                         add=True)

    def _wait_scatter(b):
        pltpu.make_async_copy(rows_v.at[b], acc_sh.at[idx_v.at[b]],
                              scat_sems[b]).wait()

    def _outer(jo, _):
        for b in range(_NBUF):
            ci = jo * _NBUF + b
            _wait_loads(ci, b)
            pltpu.sync_copy(rows_v.at[b], acc_sh.at[idx_v.at[b]], add=True)
            _start_loads(ci + _NBUF, b)
        return ()
    lax.fori_loop(0, _NFULL // _NBUF - 1, _outer, ())
    for b in range(_NBUF):
        ci = _NFULL - _NBUF + b
        _wait_loads(ci, b)
        pltpu.sync_copy(rows_v.at[b], acc_sh.at[idx_v.at[b]], add=True)

    tail_off = pl.multiple_of(base + _NFULL * _CHUNK, 16)
    pltpu.sync_copy(idx_hbm.at[pl.ds(tail_off, _TAIL)], idx_t)
    pltpu.async_copy(x_hbm.at[pl.ds(tail_off, _TAIL)],
                     rows_v.at[0, pl.ds(0, _TAIL)], tail_sem)
    pltpu.make_async_copy(x_hbm.at[pl.ds(tail_off, _TAIL)],
                          rows_v.at[0, pl.ds(0, _TAIL)], tail_sem).wait()
    pltpu.sync_copy(rows_v.at[0, pl.ds(0, _TAIL)], acc_sh.at[idx_t],
                    add=True)
    plsc.subcore_barrier()

    def _wb_desc(j):
        r0 = pl.multiple_of((j * _NS + s) * _WB_CHUNK, 64)
        return (acc_sh.at[pl.ds(r0, _WB_CHUNK)],
                out_hbm.at[c, pl.ds(r0, _WB_CHUNK)])

    rem_src = acc_sh.at[pl.ds(_N_WB_FULL * _WB_CHUNK, _N_SEG % _WB_CHUNK)]
    rem_dst = out_hbm.at[c, pl.ds(_N_WB_FULL * _WB_CHUNK,
                                  _N_SEG % _WB_CHUNK)]
    def _wb_start(j, _):
        src, dst = _wb_desc(j)
        pltpu.async_copy(src, dst, tail_sem)
        return ()
    lax.fori_loop(0, _N_WB_FULL // _NS, _wb_start, ())
    @pl.when(s < _N_WB_FULL % _NS)
    def _():
        _wb_start(_N_WB_FULL // _NS, ())
    @pl.when(s == _N_WB_FULL % _NS)
    def _():
        pltpu.async_copy(rem_src, rem_dst, tail_sem)

    def _wb_wait(j, _):
        src, dst = _wb_desc(j)
        pltpu.make_async_copy(src, dst, tail_sem).wait()
        return ()
    lax.fori_loop(0, _N_WB_FULL // _NS, _wb_wait, ())
    @pl.when(s < _N_WB_FULL % _NS)
    def _():
        _wb_wait(_N_WB_FULL // _NS, ())
    @pl.when(s == _N_WB_FULL % _NS)
    def _():
        pltpu.make_async_copy(rem_src, rem_dst, tail_sem).wait()


def _tc_add(a_ref, b_ref, o_ref):
    o_ref[...] = a_ref[0] + b_ref[0]


@jax.jit
def kernel(x, index):
    idx = index.astype(jnp.int32)
    mesh = plsc.VectorSubcoreMesh(core_axis_name="c", subcore_axis_name="s")
    partials = pl.kernel(
        _sc_body,
        out_type=jax.ShapeDtypeStruct((_NC, _N_SEG, _D), jnp.float32),
        mesh=mesh,
        scratch_types=[
            pltpu.VMEM_SHARED((_N_SEG, _D), jnp.float32),
            pltpu.VMEM((_NBUF, _CHUNK), jnp.int32),
            pltpu.VMEM((_TAIL,), jnp.int32),
            pltpu.VMEM((_NBUF, _CHUNK, _D), jnp.float32),
            *([pltpu.SemaphoreType.DMA] * (3 * _NBUF + 1)),
        ],
    )(x, idx)

    blk = 2000
    out = pl.pallas_call(
        _tc_add,
        grid=(_N_SEG // blk,),
        in_specs=[
            pl.BlockSpec((1, blk, _D), lambda i: (0, i, 0)),
            pl.BlockSpec((1, blk, _D), lambda i: (1, i, 0)),
        ],
        out_specs=pl.BlockSpec((blk, _D), lambda i: (i, 0)),
        out_shape=jax.ShapeDtypeStruct((_N_SEG, _D), jnp.float32),
    )(partials, partials)
    return out


# speedup vs baseline: 1.2397x; 1.0012x over previous
import functools

import jax
import jax.numpy as jnp
from jax import lax
from jax.experimental import pallas as pl
from jax.experimental.pallas import tpu as pltpu
from jax.experimental.pallas import tpu_sc as plsc

_N_EDGES = 320000
_D = 128
_N_SEG = 10000
_NC = 2
_NS = 16
_NW = _NC * _NS
_EDGES_PER_TILE = _N_EDGES // _NW
_CHUNK = 128
_NFULL = _EDGES_PER_TILE // _CHUNK
_TAIL = _EDGES_PER_TILE - _NFULL * _CHUNK
_NBUF = 3
_SEG_PER_TILE = _N_SEG // _NS
_WB_CHUNK = 64
_N_WB_FULL = _N_SEG // _WB_CHUNK


def _sc_body(x_hbm, idx_hbm, out_hbm, acc_sh, idx_v, idx_t, rows_v, *sems):
    row_sems = sems[:_NBUF]
    idx_sems = sems[_NBUF:2 * _NBUF]
    scat_sems = sems[2 * _NBUF:3 * _NBUF]
    tail_sem = sems[3 * _NBUF]
    c = lax.axis_index("c")
    s = lax.axis_index("s")
    tid = c * _NS + s
    base = tid * _EDGES_PER_TILE

    def _row_src(ci):
        off = pl.multiple_of(base + ci * _CHUNK, 16)
        return x_hbm.at[pl.ds(off, _CHUNK)]

    def _idx_src(ci):
        off = pl.multiple_of(base + ci * _CHUNK, 16)
        return idx_hbm.at[pl.ds(off, _CHUNK)]

    def _start_loads(ci, b):
        pltpu.async_copy(_row_src(ci), rows_v.at[b], row_sems[b])
        pltpu.async_copy(_idx_src(ci), idx_v.at[b], idx_sems[b])

    def _wait_loads(ci, b):
        pltpu.make_async_copy(_row_src(ci), rows_v.at[b], row_sems[b]).wait()
        pltpu.make_async_copy(_idx_src(ci), idx_v.at[b],
                              idx_sems[b]).wait()

    for b in range(_NBUF - 1):
        _start_loads(b, b)

    zb = _NBUF - 1
    zvec = jnp.zeros((16,), jnp.float32)
    def _zero_row(i, _):
        def _zero_lane(k, _):
            rows_v[zb, i, pl.ds(k * 16, 16)] = zvec
            return ()
        lax.fori_loop(0, _D // 16, _zero_lane, (), unroll=True)
        return ()
    lax.fori_loop(0, _CHUNK, _zero_row, ())
    seg0 = s * _SEG_PER_TILE
    rem = _SEG_PER_TILE % _CHUNK
    for j in range(_SEG_PER_TILE // _CHUNK):
        pltpu.async_copy(rows_v.at[zb],
                         acc_sh.at[pl.ds(seg0 + j * _CHUNK, _CHUNK)],
                         tail_sem)
    pltpu.async_copy(rows_v.at[zb, pl.ds(0, rem)],
                     acc_sh.at[pl.ds(seg0 + _SEG_PER_TILE - rem, rem)],
                     tail_sem)
    for j in range(_SEG_PER_TILE // _CHUNK):
        pltpu.make_async_copy(rows_v.at[zb],
                              acc_sh.at[pl.ds(seg0 + j * _CHUNK, _CHUNK)],
                              tail_sem).wait()
    pltpu.make_async_copy(rows_v.at[zb, pl.ds(0, rem)],
                          acc_sh.at[pl.ds(seg0 + _SEG_PER_TILE - rem, rem)],
                          tail_sem).wait()
    _start_loads(zb, zb)
    plsc.subcore_barrier()

    def _scatter(b):
        pltpu.async_copy(rows_v.at[b], acc_sh.at[idx_v.at[b]], scat_sems[b],
                         add=True)

    def _wait_scatter(b):
        pltpu.make_async_copy(rows_v.at[b], acc_sh.at[idx_v.at[b]],
                              scat_sems[b]).wait()

    def _outer(jo, _):
        for b in range(_NBUF):
            ci = jo * _NBUF + b
            _wait_loads(ci, b)
            pltpu.sync_copy(rows_v.at[b], acc_sh.at[idx_v.at[b]], add=True)
            _start_loads(ci + _NBUF, b)
        return ()
    lax.fori_loop(0, _NFULL // _NBUF - 1, _outer, ())
    for b in range(_NBUF):
        ci = _NFULL - _NBUF + b
        _wait_loads(ci, b)
        pltpu.sync_copy(rows_v.at[b], acc_sh.at[idx_v.at[b]], add=True)

    tail_off = pl.multiple_of(base + _NFULL * _CHUNK, 16)
    pltpu.sync_copy(idx_hbm.at[pl.ds(tail_off, _TAIL)], idx_t)
    pltpu.async_copy(x_hbm.at[pl.ds(tail_off, _TAIL)],
                     rows_v.at[0, pl.ds(0, _TAIL)], tail_sem)
    pltpu.make_async_copy(x_hbm.at[pl.ds(tail_off, _TAIL)],
                          rows_v.at[0, pl.ds(0, _TAIL)], tail_sem).wait()
    pltpu.sync_copy(rows_v.at[0, pl.ds(0, _TAIL)], acc_sh.at[idx_t],
                    add=True)
    plsc.subcore_barrier()

    def _wb_desc(j):
        r0 = pl.multiple_of((j * _NS + s) * _WB_CHUNK, 64)
        return (acc_sh.at[pl.ds(r0, _WB_CHUNK)],
                out_hbm.at[c, pl.ds(r0, _WB_CHUNK)])

    rem_src = acc_sh.at[pl.ds(_N_WB_FULL * _WB_CHUNK, _N_SEG % _WB_CHUNK)]
    rem_dst = out_hbm.at[c, pl.ds(_N_WB_FULL * _WB_CHUNK,
                                  _N_SEG % _WB_CHUNK)]
    def _wb_start(j, _):
        src, dst = _wb_desc(j)
        pltpu.async_copy(src, dst, tail_sem)
        return ()
    lax.fori_loop(0, _N_WB_FULL // _NS, _wb_start, ())
    @pl.when(s < _N_WB_FULL % _NS)
    def _():
        _wb_start(_N_WB_FULL // _NS, ())
    @pl.when(s == _N_WB_FULL % _NS)
    def _():
        pltpu.async_copy(rem_src, rem_dst, tail_sem)

    def _wb_wait(j, _):
        src, dst = _wb_desc(j)
        pltpu.make_async_copy(src, dst, tail_sem).wait()
        return ()
    lax.fori_loop(0, _N_WB_FULL // _NS, _wb_wait, ())
    @pl.when(s < _N_WB_FULL % _NS)
    def _():
        _wb_wait(_N_WB_FULL // _NS, ())
    @pl.when(s == _N_WB_FULL % _NS)
    def _():
        pltpu.make_async_copy(rem_src, rem_dst, tail_sem).wait()


def _tc_add(a_ref, b_ref, o_ref):
    o_ref[...] = a_ref[0] + b_ref[0]


@jax.jit
def kernel(x, index):
    idx = index.astype(jnp.int32)
    mesh = plsc.VectorSubcoreMesh(core_axis_name="c", subcore_axis_name="s")
    partials = pl.kernel(
        _sc_body,
        out_type=jax.ShapeDtypeStruct((_NC, _N_SEG, _D), jnp.float32),
        mesh=mesh,
        scratch_types=[
            pltpu.VMEM_SHARED((_N_SEG, _D), jnp.float32),
            pltpu.VMEM((_NBUF, _CHUNK), jnp.int32),
            pltpu.VMEM((_TAIL,), jnp.int32),
            pltpu.VMEM((_NBUF, _CHUNK, _D), jnp.float32),
            *([pltpu.SemaphoreType.DMA] * (3 * _NBUF + 1)),
        ],
    )(x, idx)

    blk = 2000
    out = pl.pallas_call(
        _tc_add,
        grid=(_N_SEG // blk,),
        in_specs=[
            pl.BlockSpec((1, blk, _D), lambda i: (0, i, 0)),
            pl.BlockSpec((1, blk, _D), lambda i: (1, i, 0)),
        ],
        out_specs=pl.BlockSpec((blk, _D), lambda i: (i, 0)),
        out_shape=jax.ShapeDtypeStruct((_N_SEG, _D), jnp.float32),
    )(partials, partials)
    return out


# speedup vs baseline: 1.3410x; 1.0817x over previous
import functools

import jax
import jax.numpy as jnp
from jax import lax
from jax.experimental import pallas as pl
from jax.experimental.pallas import tpu as pltpu
from jax.experimental.pallas import tpu_sc as plsc

_N_EDGES = 320000
_D = 128
_N_SEG = 10000
_NC = 2
_NS = 16
_NW = _NC * _NS
_EDGES_PER_TILE = _N_EDGES // _NW
_CHUNK = 64
_NFULL = _EDGES_PER_TILE // _CHUNK
_TAIL = _EDGES_PER_TILE - _NFULL * _CHUNK
_NBUF = 6
_SEG_PER_TILE = _N_SEG // _NS
_WB_CHUNK = 64
_N_WB_FULL = _N_SEG // _WB_CHUNK


def _sc_body(x_hbm, idx_hbm, out_hbm, acc_sh, idx_v, idx_t, rows_v, *sems):
    row_sems = sems[:_NBUF]
    idx_sems = sems[_NBUF:2 * _NBUF]
    scat_sems = sems[2 * _NBUF:3 * _NBUF]
    tail_sem = sems[3 * _NBUF]
    c = lax.axis_index("c")
    s = lax.axis_index("s")
    tid = c * _NS + s
    base = tid * _EDGES_PER_TILE

    def _row_src(ci):
        off = pl.multiple_of(base + ci * _CHUNK, 16)
        return x_hbm.at[pl.ds(off, _CHUNK)]

    def _idx_src(ci):
        off = pl.multiple_of(base + ci * _CHUNK, 16)
        return idx_hbm.at[pl.ds(off, _CHUNK)]

    def _start_loads(ci, b):
        pltpu.async_copy(_row_src(ci), rows_v.at[b], row_sems[b])
        pltpu.async_copy(_idx_src(ci), idx_v.at[b], idx_sems[b])

    def _wait_loads(ci, b):
        pltpu.make_async_copy(_row_src(ci), rows_v.at[b], row_sems[b]).wait()
        pltpu.make_async_copy(_idx_src(ci), idx_v.at[b],
                              idx_sems[b]).wait()

    for b in range(_NBUF - 1):
        _start_loads(b, b)

    zb = _NBUF - 1
    zvec = jnp.zeros((16,), jnp.float32)
    def _zero_row(i, _):
        def _zero_lane(k, _):
            rows_v[zb, i, pl.ds(k * 16, 16)] = zvec
            return ()
        lax.fori_loop(0, _D // 16, _zero_lane, (), unroll=True)
        return ()
    lax.fori_loop(0, _CHUNK, _zero_row, ())
    seg0 = s * _SEG_PER_TILE
    rem = _SEG_PER_TILE % _CHUNK
    for j in range(_SEG_PER_TILE // _CHUNK):
        pltpu.async_copy(rows_v.at[zb],
                         acc_sh.at[pl.ds(seg0 + j * _CHUNK, _CHUNK)],
                         tail_sem)
    pltpu.async_copy(rows_v.at[zb, pl.ds(0, rem)],
                     acc_sh.at[pl.ds(seg0 + _SEG_PER_TILE - rem, rem)],
                     tail_sem)
    for j in range(_SEG_PER_TILE // _CHUNK):
        pltpu.make_async_copy(rows_v.at[zb],
                              acc_sh.at[pl.ds(seg0 + j * _CHUNK, _CHUNK)],
                              tail_sem).wait()
    pltpu.make_async_copy(rows_v.at[zb, pl.ds(0, rem)],
                          acc_sh.at[pl.ds(seg0 + _SEG_PER_TILE - rem, rem)],
                          tail_sem).wait()
    _start_loads(zb, zb)
    plsc.subcore_barrier()

    def _scatter(b):
        pltpu.async_copy(rows_v.at[b], acc_sh.at[idx_v.at[b]], scat_sems[b],
                         add=True)

    def _wait_scatter(b):
        pltpu.make_async_copy(rows_v.at[b], acc_sh.at[idx_v.at[b]],
                              scat_sems[b]).wait()

    for k in (0, 1):
        _wait_loads(k, k % _NBUF)
        _scatter(k % _NBUF)
    def _outer(jo, _):
        for b in range(_NBUF):
            k = jo * _NBUF + b + 2
            bk = (b + 2) % _NBUF
            _wait_loads(k, bk)
            _scatter(bk)
            bws = b % _NBUF
            _wait_scatter(bws)
            _start_loads(k + 4, bws)
        return ()
    lax.fori_loop(0, (_NFULL - _NBUF) // _NBUF, _outer, ())
    for k in range(_NFULL - 4, _NFULL):
        _wait_loads(k, k % _NBUF)
        _scatter(k % _NBUF)
        _wait_scatter((k - 2) % _NBUF)
    for k in (_NFULL - 2, _NFULL - 1):
        _wait_scatter(k % _NBUF)

    tail_off = pl.multiple_of(base + _NFULL * _CHUNK, 16)
    pltpu.sync_copy(idx_hbm.at[pl.ds(tail_off, _TAIL)], idx_t)
    pltpu.async_copy(x_hbm.at[pl.ds(tail_off, _TAIL)],
                     rows_v.at[0, pl.ds(0, _TAIL)], tail_sem)
    pltpu.make_async_copy(x_hbm.at[pl.ds(tail_off, _TAIL)],
                          rows_v.at[0, pl.ds(0, _TAIL)], tail_sem).wait()
    pltpu.sync_copy(rows_v.at[0, pl.ds(0, _TAIL)], acc_sh.at[idx_t],
                    add=True)
    plsc.subcore_barrier()

    def _wb_desc(j):
        r0 = pl.multiple_of((j * _NS + s) * _WB_CHUNK, 64)
        return (acc_sh.at[pl.ds(r0, _WB_CHUNK)],
                out_hbm.at[c, pl.ds(r0, _WB_CHUNK)])

    rem_src = acc_sh.at[pl.ds(_N_WB_FULL * _WB_CHUNK, _N_SEG % _WB_CHUNK)]
    rem_dst = out_hbm.at[c, pl.ds(_N_WB_FULL * _WB_CHUNK,
                                  _N_SEG % _WB_CHUNK)]
    def _wb_start(j, _):
        src, dst = _wb_desc(j)
        pltpu.async_copy(src, dst, tail_sem)
        return ()
    lax.fori_loop(0, _N_WB_FULL // _NS, _wb_start, ())
    @pl.when(s < _N_WB_FULL % _NS)
    def _():
        _wb_start(_N_WB_FULL // _NS, ())
    @pl.when(s == _N_WB_FULL % _NS)
    def _():
        pltpu.async_copy(rem_src, rem_dst, tail_sem)

    def _wb_wait(j, _):
        src, dst = _wb_desc(j)
        pltpu.make_async_copy(src, dst, tail_sem).wait()
        return ()
    lax.fori_loop(0, _N_WB_FULL // _NS, _wb_wait, ())
    @pl.when(s < _N_WB_FULL % _NS)
    def _():
        _wb_wait(_N_WB_FULL // _NS, ())
    @pl.when(s == _N_WB_FULL % _NS)
    def _():
        pltpu.make_async_copy(rem_src, rem_dst, tail_sem).wait()


def _tc_add(a_ref, b_ref, o_ref):
    o_ref[...] = a_ref[0] + b_ref[0]


@jax.jit
def kernel(x, index):
    idx = index.astype(jnp.int32)
    mesh = plsc.VectorSubcoreMesh(core_axis_name="c", subcore_axis_name="s")
    partials = pl.kernel(
        _sc_body,
        out_type=jax.ShapeDtypeStruct((_NC, _N_SEG, _D), jnp.float32),
        mesh=mesh,
        scratch_types=[
            pltpu.VMEM_SHARED((_N_SEG, _D), jnp.float32),
            pltpu.VMEM((_NBUF, _CHUNK), jnp.int32),
            pltpu.VMEM((_TAIL,), jnp.int32),
            pltpu.VMEM((_NBUF, _CHUNK, _D), jnp.float32),
            *([pltpu.SemaphoreType.DMA] * (3 * _NBUF + 1)),
        ],
    )(x, idx)

    blk = 2000
    out = pl.pallas_call(
        _tc_add,
        grid=(_N_SEG // blk,),
        in_specs=[
            pl.BlockSpec((1, blk, _D), lambda i: (0, i, 0)),
            pl.BlockSpec((1, blk, _D), lambda i: (1, i, 0)),
        ],
        out_specs=pl.BlockSpec((blk, _D), lambda i: (i, 0)),
        out_shape=jax.ShapeDtypeStruct((_N_SEG, _D), jnp.float32),
    )(partials, partials)
    return out


# speedup vs baseline: 1.3415x; 1.0004x over previous
import jax
import jax.numpy as jnp
from jax import lax
from jax.experimental import pallas as pl
from jax.experimental.pallas import tpu as pltpu
from jax.experimental.pallas import tpu_sc as plsc

_N_EDGES = 320000
_D = 128
_N_SEG = 10000
_NC = 2
_NS = 16
_NW = _NC * _NS
_EDGES_PER_TILE = _N_EDGES // _NW
_CHUNK = 64
_NFULL = _EDGES_PER_TILE // _CHUNK
_TAIL = _EDGES_PER_TILE - _NFULL * _CHUNK
_NBUF = 6
_SEG_PER_TILE = _N_SEG // _NS
_WB_CHUNK = 64
_N_WB_FULL = _N_SEG // _WB_CHUNK


def _sc_body(x_hbm, idx_hbm, out_hbm, acc_sh, idx_v, idx_t, rows_v, *sems):
    row_sems = sems[:_NBUF]
    idx_sems = sems[_NBUF:2 * _NBUF]
    scat_sems = sems[2 * _NBUF:3 * _NBUF]
    tail_sem = sems[3 * _NBUF]
    c = lax.axis_index("c")
    s = lax.axis_index("s")
    tid = c * _NS + s
    base = tid * _EDGES_PER_TILE

    def _row_src(ci):
        off = pl.multiple_of(base + ci * _CHUNK, 16)
        return x_hbm.at[pl.ds(off, _CHUNK)]

    def _idx_src(ci):
        off = pl.multiple_of(base + ci * _CHUNK, 16)
        return idx_hbm.at[pl.ds(off, _CHUNK)]

    def _start_loads(ci, b):
        pltpu.async_copy(_row_src(ci), rows_v.at[b], row_sems[b])
        pltpu.async_copy(_idx_src(ci), idx_v.at[b], idx_sems[b])

    def _wait_loads(ci, b):
        pltpu.make_async_copy(_row_src(ci), rows_v.at[b], row_sems[b]).wait()
        pltpu.make_async_copy(_idx_src(ci), idx_v.at[b],
                              idx_sems[b]).wait()

    for b in range(_NBUF - 1):
        _start_loads(b, b)

    zb = _NBUF - 1
    zvec = jnp.zeros((16,), jnp.float32)
    def _zero_row(i, _):
        def _zero_lane(k, _):
            rows_v[zb, i, pl.ds(k * 16, 16)] = zvec
            return ()
        lax.fori_loop(0, _D // 16, _zero_lane, (), unroll=True)
        return ()
    lax.fori_loop(0, _CHUNK, _zero_row, ())
    seg0 = s * _SEG_PER_TILE
    rem = _SEG_PER_TILE % _CHUNK
    for j in range(_SEG_PER_TILE // _CHUNK):
        pltpu.async_copy(rows_v.at[zb],
                         acc_sh.at[pl.ds(seg0 + j * _CHUNK, _CHUNK)],
                         tail_sem)
    pltpu.async_copy(rows_v.at[zb, pl.ds(0, rem)],
                     acc_sh.at[pl.ds(seg0 + _SEG_PER_TILE - rem, rem)],
                     tail_sem)
    for j in range(_SEG_PER_TILE // _CHUNK):
        pltpu.make_async_copy(rows_v.at[zb],
                              acc_sh.at[pl.ds(seg0 + j * _CHUNK, _CHUNK)],
                              tail_sem).wait()
    pltpu.make_async_copy(rows_v.at[zb, pl.ds(0, rem)],
                          acc_sh.at[pl.ds(seg0 + _SEG_PER_TILE - rem, rem)],
                          tail_sem).wait()
    _start_loads(zb, zb)
    plsc.subcore_barrier()

    def _scatter(b):
        pltpu.async_copy(rows_v.at[b], acc_sh.at[idx_v.at[b]], scat_sems[b],
                         add=True)

    def _wait_scatter(b):
        pltpu.make_async_copy(rows_v.at[b], acc_sh.at[idx_v.at[b]],
                              scat_sems[b]).wait()

    for k in (0, 1):
        _wait_loads(k, k % _NBUF)
        _scatter(k % _NBUF)
    def _outer(jo, _):
        for b in range(_NBUF):
            k = jo * _NBUF + b + 2
            bk = (b + 2) % _NBUF
            _wait_loads(k, bk)
            _scatter(bk)
            bws = b % _NBUF
            _wait_scatter(bws)
            _start_loads(k + 4, bws)
        return ()
    lax.fori_loop(0, (_NFULL - _NBUF) // _NBUF, _outer, ())
    for k in range(_NFULL - 4, _NFULL):
        _wait_loads(k, k % _NBUF)
        _scatter(k % _NBUF)
        _wait_scatter((k - 2) % _NBUF)
    for k in (_NFULL - 2, _NFULL - 1):
        _wait_scatter(k % _NBUF)

    tail_off = pl.multiple_of(base + _NFULL * _CHUNK, 16)
    pltpu.sync_copy(idx_hbm.at[pl.ds(tail_off, _TAIL)], idx_t)
    pltpu.async_copy(x_hbm.at[pl.ds(tail_off, _TAIL)],
                     rows_v.at[0, pl.ds(0, _TAIL)], tail_sem)
    pltpu.make_async_copy(x_hbm.at[pl.ds(tail_off, _TAIL)],
                          rows_v.at[0, pl.ds(0, _TAIL)], tail_sem).wait()
    pltpu.sync_copy(rows_v.at[0, pl.ds(0, _TAIL)], acc_sh.at[idx_t],
                    add=True)
    plsc.subcore_barrier()

    def _wb_desc(j):
        r0 = pl.multiple_of((j * _NS + s) * _WB_CHUNK, 64)
        return (acc_sh.at[pl.ds(r0, _WB_CHUNK)],
                out_hbm.at[c, pl.ds(r0, _WB_CHUNK)])

    rem_src = acc_sh.at[pl.ds(_N_WB_FULL * _WB_CHUNK, _N_SEG % _WB_CHUNK)]
    rem_dst = out_hbm.at[c, pl.ds(_N_WB_FULL * _WB_CHUNK,
                                  _N_SEG % _WB_CHUNK)]
    def _wb_start(j, _):
        src, dst = _wb_desc(j)
        pltpu.async_copy(src, dst, tail_sem)
        return ()
    lax.fori_loop(0, _N_WB_FULL // _NS, _wb_start, ())
    @pl.when(s < _N_WB_FULL % _NS)
    def _():
        _wb_start(_N_WB_FULL // _NS, ())
    @pl.when(s == _N_WB_FULL % _NS)
    def _():
        pltpu.async_copy(rem_src, rem_dst, tail_sem)

    def _wb_wait(j, _):
        src, dst = _wb_desc(j)
        pltpu.make_async_copy(src, dst, tail_sem).wait()
        return ()
    lax.fori_loop(0, _N_WB_FULL // _NS, _wb_wait, ())
    @pl.when(s < _N_WB_FULL % _NS)
    def _():
        _wb_wait(_N_WB_FULL // _NS, ())
    @pl.when(s == _N_WB_FULL % _NS)
    def _():
        pltpu.make_async_copy(rem_src, rem_dst, tail_sem).wait()


def _tc_add(a_ref, b_ref, o_ref):
    o_ref[...] = a_ref[0] + b_ref[0]


@jax.jit
def kernel(x, index):
    idx = index.astype(jnp.int32)
    mesh = plsc.VectorSubcoreMesh(core_axis_name="c", subcore_axis_name="s")
    partials = pl.kernel(
        _sc_body,
        out_type=jax.ShapeDtypeStruct((_NC, _N_SEG, _D), jnp.float32),
        mesh=mesh,
        scratch_types=[
            pltpu.VMEM_SHARED((_N_SEG, _D), jnp.float32),
            pltpu.VMEM((_NBUF, _CHUNK), jnp.int32),
            pltpu.VMEM((_TAIL,), jnp.int32),
            pltpu.VMEM((_NBUF, _CHUNK, _D), jnp.float32),
            *([pltpu.SemaphoreType.DMA] * (3 * _NBUF + 1)),
        ],
    )(x, idx)

    blk = 2000
    out = pl.pallas_call(
        _tc_add,
        grid=(_N_SEG // blk,),
        in_specs=[
            pl.BlockSpec((1, blk, _D), lambda i: (0, i, 0)),
            pl.BlockSpec((1, blk, _D), lambda i: (1, i, 0)),
        ],
        out_specs=pl.BlockSpec((blk, _D), lambda i: (i, 0)),
        out_shape=jax.ShapeDtypeStruct((_N_SEG, _D), jnp.float32),
    )(partials, partials)
    return out
